# baseline (speedup 1.0000x reference)
import functools

import jax
import jax.numpy as jnp
from jax import lax
from jax.experimental import pallas as pl
from jax.experimental.pallas import tpu as pltpu
from jax.experimental.pallas import tpu_sc as plsc

B = 16384
HIST = 50
PD = 64
HID = 128
NC = 2
NS = 16
NW = NC * NS
S = B // NW
SPS = 2
NBUF = 4
NP = S // SPS
L = 16
NG = PD // L
TG = (HIST + L - 1) // L


def _sc_pool(hist2, pids, table, w64, b16):
    mesh = plsc.VectorSubcoreMesh(core_axis_name="c", subcore_axis_name="s")

    @functools.partial(
        pl.kernel,
        out_type=(jax.ShapeDtypeStruct((B, PD), jnp.float32),
                  jax.ShapeDtypeStruct((B, PD), jnp.float32),
                  jax.ShapeDtypeStruct((B, L), jnp.float32)),
        mesh=mesh,
        compiler_params=pltpu.CompilerParams(needs_layout_passes=False,
                                             use_tc_tiling_on_sc=False),
        scratch_types=[
            pltpu.VMEM((NP, SPS * HIST), jnp.int32),
            pltpu.VMEM((S,), jnp.int32),
            pltpu.VMEM((128, PD), jnp.float32),
            pltpu.VMEM((NBUF, SPS * HIST, PD), jnp.float32),
            pltpu.VMEM((S, PD), jnp.float32),
            pltpu.VMEM((S, L), jnp.float32),
            pltpu.VMEM((SPS, L, L * TG), jnp.float32),
            pltpu.VMEM((PD,), jnp.float32),
            pltpu.VMEM((L,), jnp.float32),
            pltpu.SemaphoreType.DMA((NBUF,)),
            pltpu.SemaphoreType.DMA,
        ],
    )
    def k(hist_hbm, pid_hbm, table_hbm, w_hbm, b_hbm,
          pooled_hbm, prodrows_hbm, den_hbm,
          hidx_v, pidx_v, prow_v, ring_v, pool_v, den_v, a_v, w_v, b_v,
          sems, gsem):
        cid = lax.axis_index("c")
        sid = lax.axis_index("s")
        wid = cid * NS + sid
        base = wid * S

        pltpu.sync_copy(w_hbm, w_v)
        pltpu.sync_copy(b_hbm, b_v)
        pltpu.sync_copy(hist_hbm.at[pl.ds(wid * NP, NP)], hidx_v)
        pltpu.sync_copy(pid_hbm.at[pl.ds(base, S)], pidx_v)

        for kk in range(S // 128):
            pltpu.async_copy(
                table_hbm.at[pidx_v.at[pl.ds(kk * 128, 128)]], prow_v, gsem
            ).wait()
            pltpu.sync_copy(prow_v, prodrows_hbm.at[pl.ds(base + kk * 128, 128)])

        def pair_copy(p, slot):
            return pltpu.make_async_copy(
                table_hbm.at[hidx_v.at[p]], ring_v.at[slot], sems.at[slot])

        for u in range(NBUF):
            pair_copy(u, u).start()

        w_vecs = [w_v[pl.ds(c * L, L)] for c in range(NG)]
        b_vec = b_v[...]
        iota = lax.iota(jnp.int32, L)
        pad_mask = iota < jnp.full((L,), HIST - (TG - 1) * L, jnp.int32)

        @pl.loop(0, NP)
        def _(p):
            u = lax.rem(p, NBUF)
            pair_copy(p, u).wait()
            hs = [ring_v.at[u, pl.ds(sloc * HIST, HIST)] for sloc in range(SPS)]
            for sloc in range(SPS):
                h = hs[sloc]
                for t in range(HIST):
                    acc = (h[t, pl.ds(0, L)] * w_vecs[0]
                           + h[t, pl.ds(L, L)] * w_vecs[1]) \
                        + (h[t, pl.ds(2 * L, L)] * w_vecs[2]
                           + h[t, pl.ds(3 * L, L)] * w_vecs[3])
                    plsc.store_scatter(
                        a_v.at[sloc], [iota, jnp.full((L,), t, jnp.int32)], acc)
            evs = []
            for sloc in range(SPS):
                evecs = []
                den = None
                for g in range(TG):
                    rows = [a_v[sloc, j, pl.ds(g * L, L)] for j in range(L)]
                    while len(rows) > 1:
                        rows = [rows[2 * j] + rows[2 * j + 1]
                                for j in range(len(rows) // 2)]
                    sv = rows[0] + b_vec
                    uu = jnp.exp(sv * 2.0)
                    e = jnp.exp(1.0 - 2.0 / (uu + 1.0))
                    if g == TG - 1:
                        e = jnp.where(pad_mask, e, 0.0)
                    evecs.append(e)
                    den = e if den is None else den + e
                evs.append(evecs)
                den_v[p * SPS + sloc, pl.ds(0, L)] = den
            for sloc in range(SPS):
                h = hs[sloc]
                evecs = evs[sloc]
                num = [jnp.zeros((L,), jnp.float32) for _ in range(NG)]
                for t in range(HIST):
                    e_t = jnp.full((L,), evecs[t // L][t % L], jnp.float32)
                    for c in range(NG):
                        num[c] = num[c] + e_t * h[t, pl.ds(c * L, L)]
                i = p * SPS + sloc
                for c in range(NG):
                    pool_v[i, pl.ds(c * L, L)] = num[c]

            @pl.when(p + NBUF < NP)
            def _():
                pair_copy(p + NBUF, u).start()

        pltpu.sync_copy(pool_v, pooled_hbm.at[pl.ds(base, S)])
        pltpu.sync_copy(den_v, den_hbm.at[pl.ds(base, S)])

    return k(hist2, pids, table, w64, b16)


def _tc_mlp(pooled, prod, den, w1a, w1b, b1, w2, b2):
    BS = 512

    def body(p_ref, q_ref, d_ref, w1a_ref, w1b_ref, b1_ref, w2_ref, b2_ref,
             o_ref):
        dtot = jnp.sum(d_ref[...], axis=1, keepdims=True)
        p_norm = p_ref[...] / dtot
        x = jnp.dot(p_norm, w1a_ref[...], preferred_element_type=jnp.float32)
        x = x + jnp.dot(q_ref[...], w1b_ref[...], preferred_element_type=jnp.float32)
        x = jnp.maximum(x + b1_ref[...], 0.0)
        y = jnp.dot(x, w2_ref[...], preferred_element_type=jnp.float32) + b2_ref[...]
        o_ref[...] = jax.nn.sigmoid(y)

    return pl.pallas_call(
        body,
        grid=(B // BS,),
        in_specs=[
            pl.BlockSpec((BS, PD), lambda i: (i, 0)),
            pl.BlockSpec((BS, PD), lambda i: (i, 0)),
            pl.BlockSpec((BS, L), lambda i: (i, 0)),
            pl.BlockSpec((PD, HID), lambda i: (0, 0)),
            pl.BlockSpec((PD, HID), lambda i: (0, 0)),
            pl.BlockSpec((1, HID), lambda i: (0, 0)),
            pl.BlockSpec((HID, 1), lambda i: (0, 0)),
            pl.BlockSpec((1, 1), lambda i: (0, 0)),
        ],
        out_specs=pl.BlockSpec((BS, 1), lambda i: (i, 0)),
        out_shape=jax.ShapeDtypeStruct((B, 1), jnp.float32),
    )(pooled, prod, den, w1a, w1b, b1, w2, b2)


def kernel(user_ids, product_ids, user_history, user_table, prod_table,
           attn_W, attn_b, mlp1_W, mlp1_b, mlp2_W, mlp2_b):
    hist2 = user_history.astype(jnp.int32).reshape(B // SPS, SPS * HIST)
    pids = product_ids.astype(jnp.int32)
    w64 = attn_W.reshape(PD).astype(jnp.float32)
    b16 = jnp.broadcast_to(attn_b.reshape(1).astype(jnp.float32), (L,))
    pooled, prodrows, den = _sc_pool(hist2, pids, prod_table, w64, b16)
    out = _tc_mlp(pooled, prodrows, den,
                  mlp1_W[:PD], mlp1_W[PD:],
                  mlp1_b.reshape(1, HID), mlp2_W, mlp2_b.reshape(1, 1))
    return out


# speedup vs baseline: 1.0883x; 1.0883x over previous
import functools

import jax
import jax.numpy as jnp
from jax import lax
from jax.experimental import pallas as pl
from jax.experimental.pallas import tpu as pltpu
from jax.experimental.pallas import tpu_sc as plsc

B = 16384
HIST = 50
PD = 64
HID = 128
NC = 2
NS = 16
NW = NC * NS
S = B // NW
SPS = 2
NBUF = 6
NFLY = 4
NP = S // SPS
L = 16


def _sc_gather(hist2, pids, table):
    mesh = plsc.VectorSubcoreMesh(core_axis_name="c", subcore_axis_name="s")

    @functools.partial(
        pl.kernel,
        out_type=(jax.ShapeDtypeStruct((B * HIST, PD), jnp.float32),
                  jax.ShapeDtypeStruct((B, PD), jnp.float32)),
        mesh=mesh,
        compiler_params=pltpu.CompilerParams(needs_layout_passes=False,
                                             use_tc_tiling_on_sc=False),
        scratch_types=[
            pltpu.VMEM((NP, SPS * HIST), jnp.int32),
            pltpu.VMEM((S,), jnp.int32),
            pltpu.VMEM((128, PD), jnp.float32),
            pltpu.VMEM((NBUF, SPS * HIST, PD), jnp.float32),
            pltpu.SemaphoreType.DMA((NBUF,)),
            pltpu.SemaphoreType.DMA((NBUF,)),
            pltpu.SemaphoreType.DMA,
        ],
    )
    def k(hist_hbm, pid_hbm, table_hbm, histout_hbm, prodout_hbm,
          hidx_v, pidx_v, prow_v, ring_v, gsems, wsems, gsem):
        cid = lax.axis_index("c")
        sid = lax.axis_index("s")
        wid = cid * NS + sid
        base = wid * S

        pltpu.sync_copy(hist_hbm.at[pl.ds(wid * NP, NP)], hidx_v)
        pltpu.sync_copy(pid_hbm.at[pl.ds(base, S)], pidx_v)

        for kk in range(S // 128):
            pltpu.async_copy(
                table_hbm.at[pidx_v.at[pl.ds(kk * 128, 128)]], prow_v, gsem
            ).wait()
            pltpu.sync_copy(prow_v, prodout_hbm.at[pl.ds(base + kk * 128, 128)])

        def gcopy(p, slot):
            return pltpu.make_async_copy(
                table_hbm.at[hidx_v.at[p]], ring_v.at[slot], gsems.at[slot])

        def wcopy(p, slot):
            return pltpu.make_async_copy(
                ring_v.at[slot],
                histout_hbm.at[pl.ds((wid * NP + p) * SPS * HIST, SPS * HIST)],
                wsems.at[slot])

        for j in range(NFLY):
            gcopy(j, j).start()

        @pl.loop(0, NP)
        def _(p):
            u = lax.rem(p, NBUF)
            gcopy(p, u).wait()
            wcopy(p, u).start()
            nxt = p + NFLY
            v = lax.rem(nxt, NBUF)

            @pl.when(jnp.logical_and(nxt < NP, nxt >= NBUF))
            def _():
                wcopy(nxt, v).wait()
                gcopy(nxt, v).start()

            @pl.when(jnp.logical_and(nxt < NP, nxt < NBUF))
            def _():
                gcopy(nxt, v).start()

        for u in range(NBUF):
            wcopy(0, u).wait()

    return k(hist2, pids, table)


def _tc_attn_mlp(hist3, prod, w3, b_s, w1a, w1b, b1, w2, b2):
    BS = 512

    def body(h_ref, q_ref, w3_ref, bs_ref, w1a_ref, w1b_ref, b1_ref,
             w2_ref, b2_ref, o_ref):
        h = h_ref[...]
        s = jnp.sum(h * w3_ref[...], axis=2) + bs_ref[...]
        e = jnp.exp(jnp.tanh(s))
        den = jnp.sum(e, axis=1, keepdims=True)
        pooled = jnp.sum(h * e[:, :, None], axis=1) / den
        x = jnp.dot(pooled, w1a_ref[...], preferred_element_type=jnp.float32)
        x = x + jnp.dot(q_ref[...], w1b_ref[...],
                        preferred_element_type=jnp.float32)
        x = jnp.maximum(x + b1_ref[...], 0.0)
        y = jnp.dot(x, w2_ref[...], preferred_element_type=jnp.float32)
        o_ref[...] = jax.nn.sigmoid(y + b2_ref[...])

    return pl.pallas_call(
        body,
        grid=(B // BS,),
        in_specs=[
            pl.BlockSpec((BS, HIST, PD), lambda i: (i, 0, 0)),
            pl.BlockSpec((BS, PD), lambda i: (i, 0)),
            pl.BlockSpec((1, 1, PD), lambda i: (0, 0, 0)),
            pl.BlockSpec((1, 1), lambda i: (0, 0)),
            pl.BlockSpec((PD, HID), lambda i: (0, 0)),
            pl.BlockSpec((PD, HID), lambda i: (0, 0)),
            pl.BlockSpec((1, HID), lambda i: (0, 0)),
            pl.BlockSpec((HID, 1), lambda i: (0, 0)),
            pl.BlockSpec((1, 1), lambda i: (0, 0)),
        ],
        out_specs=pl.BlockSpec((BS, 1), lambda i: (i, 0)),
        out_shape=jax.ShapeDtypeStruct((B, 1), jnp.float32),
    )(hist3, prod, w3, b_s, w1a, w1b, b1, w2, b2)


def kernel(user_ids, product_ids, user_history, user_table, prod_table,
           attn_W, attn_b, mlp1_W, mlp1_b, mlp2_W, mlp2_b):
    hist2 = user_history.astype(jnp.int32).reshape(B // SPS, SPS * HIST)
    pids = product_ids.astype(jnp.int32)
    hist_rows, prodrows = _sc_gather(hist2, pids, prod_table)
    hist3 = hist_rows.reshape(B, HIST, PD)
    out = _tc_attn_mlp(hist3, prodrows,
                       attn_W.reshape(1, 1, PD),
                       attn_b.reshape(1, 1),
                       mlp1_W[:PD], mlp1_W[PD:],
                       mlp1_b.reshape(1, HID), mlp2_W, mlp2_b.reshape(1, 1))
    return out
